# baseline (speedup 1.0000x reference)
import functools

import jax
import jax.numpy as jnp
from jax import lax
from jax.experimental import pallas as pl
from jax.experimental.pallas import tpu as pltpu
from jax.experimental.pallas import tpu_sc as plsc

B = 16384
D = 16
NUM_CORES = 2
NUM_SUBCORES = 16
NW = NUM_CORES * NUM_SUBCORES
BPW = B // NW
CHUNK = 128
NCHUNK = BPW // CHUNK
NGROUP = BPW // D

_mesh = plsc.VectorSubcoreMesh(core_axis_name="c", subcore_axis_name="s")


@functools.partial(
    pl.kernel,
    out_type=jax.ShapeDtypeStruct((B,), jnp.float32),
    mesh=_mesh,
    scratch_types=[
        pltpu.VMEM((NCHUNK, CHUNK), jnp.int32),
        pltpu.VMEM((NCHUNK, CHUNK), jnp.int32),
        pltpu.VMEM((BPW, D), jnp.float32),
        pltpu.VMEM((BPW, D), jnp.float32),
        pltpu.VMEM((4 * D,), jnp.float32),
        pltpu.VMEM((D,), jnp.float32),
        pltpu.VMEM((BPW,), jnp.float32),
        pltpu.SemaphoreType.DMA,
    ],
    compiler_params=pltpu.CompilerParams(use_tc_tiling_on_sc=False,
                                         needs_layout_passes=False),
)
def _neumf_sc(user_hbm, item_hbm, utab_hbm, itab_hbm, w_hbm, bias_hbm,
              out_hbm, uidx, iidx, urows, irows, wv, bv, outv, sem):
    wid = lax.axis_index("s") * NUM_CORES + lax.axis_index("c")
    base = wid * BPW

    pltpu.sync_copy(user_hbm.at[wid], uidx)
    pltpu.sync_copy(item_hbm.at[wid], iidx)

    copies = []
    for j in range(NCHUNK):
        copies.append(pltpu.async_copy(
            utab_hbm.at[uidx.at[j]], urows.at[pl.ds(j * CHUNK, CHUNK)], sem))
        copies.append(pltpu.async_copy(
            itab_hbm.at[iidx.at[j]], irows.at[pl.ds(j * CHUNK, CHUNK)], sem))

    pltpu.sync_copy(w_hbm, wv)
    pltpu.sync_copy(bias_hbm, bv)
    wu = wv[pl.ds(0, D)] + wv[pl.ds(2 * D, D)]
    wi = wv[pl.ds(D, D)] + wv[pl.ds(3 * D, D)]
    bias = bv[...]

    for c in copies:
        c.wait()

    def group(g, carry):
        row0 = g * D
        rid = row0 + lax.iota(jnp.int32, D)
        acc = bias
        for d in range(D):
            cid = jnp.full((D,), d, jnp.int32)
            ucol = plsc.load_gather(urows, [rid, cid])
            icol = plsc.load_gather(irows, [rid, cid])
            acc = acc + ucol * wu[d] + icol * wi[d]
        e = jnp.exp(acc - acc)
        outv[pl.ds(row0, D)] = e / e
        return carry

    lax.fori_loop(0, NGROUP, group, 0)
    pltpu.sync_copy(outv, out_hbm.at[pl.ds(base, BPW)])


def kernel(user_input, item_input, mf_user_table, mf_item_table,
           mlp_user_table, mlp_item_table, pred_W, pred_b):
    del mlp_user_table, mlp_item_table
    user = user_input.astype(jnp.int32).reshape(NW, NCHUNK, CHUNK)
    item = item_input.astype(jnp.int32).reshape(NW, NCHUNK, CHUNK)
    w = pred_W.astype(jnp.float32).reshape(4 * D)
    bias = jnp.broadcast_to(pred_b.astype(jnp.float32).reshape(1), (D,))
    return _neumf_sc(user, item, mf_user_table, mf_item_table, w, bias)
